# baseline (speedup 1.0000x reference)
import jax
import jax.numpy as jnp
from jax.experimental import pallas as pl
from jax.experimental.pallas import tpu as pltpu
from jax.experimental.compute_on import compute_on


def _tc_loss_kernel(pred_ref, gt_ref, out_ref, acc_ref):
    i = pl.program_id(0)

    @pl.when(i == 0)
    def _():
        acc_ref[0] = 0.0

    g = gt_ref[...]
    d = pred_ref[...] - g
    mask = (g[:, 0:1] != -1.0).astype(jnp.float32)
    acc_ref[0] += jnp.sum(d * d * mask)

    @pl.when(i == pl.num_programs(0) - 1)
    def _():
        out_ref[0, 0] = acc_ref[0]


def _tc_partial(pred2, gt2, n_rows, block_rows):
    rows, C = pred2.shape
    grid = n_rows // block_rows
    out = pl.pallas_call(
        _tc_loss_kernel,
        grid=(grid,),
        in_specs=[
            pl.BlockSpec((block_rows, C), lambda i: (i, 0)),
            pl.BlockSpec((block_rows, C), lambda i: (i, 0)),
        ],
        out_specs=pl.BlockSpec((1, 1), lambda i: (0, 0), memory_space=pltpu.SMEM),
        out_shape=jax.ShapeDtypeStruct((1, 1), jnp.float32),
        scratch_shapes=[pltpu.SMEM((1,), jnp.float32)],
    )(pred2, gt2)
    return out[0, 0]


_SC_ROWS = 12288
_TC_BLOCK_ROWS = 2048


def kernel(pred, gt):
    B, N, C = pred.shape
    rows = B * N
    pred2 = pred.reshape(rows, C)
    gt2 = gt.reshape(rows, C)
    tc_rows = rows - _SC_ROWS

    @compute_on("tpu_sparsecore")
    @jax.jit
    def sc_tail(p, g):
        ps = jax.lax.slice_in_dim(p, tc_rows, rows, axis=0)
        gs = jax.lax.slice_in_dim(g, tc_rows, rows, axis=0)
        d = ps - gs
        m = (gs[:, 0:1] != -1.0).astype(jnp.float32)
        return jnp.sum(d * d * m)

    sc_part = sc_tail(pred2, gt2)
    tc_part = _tc_partial(pred2, gt2, tc_rows, _TC_BLOCK_ROWS)
    return tc_part + sc_part
